# Initial kernel scaffold; baseline (speedup 1.0000x reference)
import jax
import jax.numpy as jnp
from jax.experimental import pallas as pl


def kernel(x, gamma, beta):
    raise NotImplementedError("write your pallas kernel here")


# speedup vs baseline: 673.6593x; 673.6593x over previous
import functools

import jax
import jax.numpy as jnp
import numpy as np
from jax import lax
from jax.experimental import pallas as pl
from jax.experimental.pallas import tpu as pltpu
from jax.experimental.pallas import tpu_sc as plsc

B = 8
C = 4
L = 64
N = 90
BIN = 360 // N
V = L * L * L
NW = 32
NBINS_PAD = 8128
CH = 16384
NCHUNK = V // CH
LANES = 16


def _build_bin_table() -> np.ndarray:
    i3, i4, i5 = np.meshgrid(
        np.arange(L), np.arange(L), np.arange(L), indexing="ij"
    )
    i3 = i3.ravel().astype(np.float32)
    i4 = i4.ravel().astype(np.float32)
    i5 = i5.ravel().astype(np.float32)
    x_comp = np.sqrt(i4 * i4 + i5 * i5, dtype=np.float32)
    z_comp = np.sqrt(i4 * i4 + i3 * i3, dtype=np.float32)
    deg = np.float32(180.0)
    pi = np.float32(np.pi)
    binf = np.float32(BIN)
    ax = (np.arctan2(x_comp, i3) * deg / pi / binf).astype(np.int32)
    az = (np.arctan2(i5, z_comp) * deg / pi / binf).astype(np.int32)
    return (ax * np.int32(N) + az).astype(np.int32)


_BIN_TABLE = _build_bin_table()


def _sc_body(x_hbm, tbl_hbm, out_hbm, xbuf, tbuf, acc):
    wid = lax.axis_index("s") * 2 + lax.axis_index("c")
    xrow = (wid & 7) * C + lax.shift_right_logical(wid, 3)

    zero16 = jnp.zeros((LANES,), jnp.int32)

    def _zero(i, _):
        acc[pl.ds(i * LANES, LANES)] = zero16
        return _

    lax.fori_loop(0, NBINS_PAD // LANES, _zero, None)

    iota = lax.iota(jnp.int32, LANES)

    for chunk in range(NCHUNK):
        base0 = chunk * CH
        pltpu.sync_copy(x_hbm.at[xrow, pl.ds(base0, CH)], xbuf)
        pltpu.sync_copy(tbl_hbm.at[pl.ds(base0, CH)], tbuf)

        def _step(i, _, base0=base0):
            off = i * LANES
            xv = xbuf[pl.ds(off, LANES)]
            iv = tbuf[pl.ds(off, LANES)]
            code = iota + (base0 + off + 1)
            plsc.store_scatter(acc, [iv], code, mask=xv != 0.0)
            return _

        lax.fori_loop(0, CH // LANES, _step, None)

    pltpu.sync_copy(acc, out_hbm.at[wid])


@functools.partial(jax.jit, static_argnames=())
def _sc_scatter(x2, tbl):
    mesh = plsc.VectorSubcoreMesh(core_axis_name="c", subcore_axis_name="s")
    kern = pl.kernel(
        _sc_body,
        out_type=jax.ShapeDtypeStruct((NW, NBINS_PAD), jnp.int32),
        mesh=mesh,
        scratch_types=[
            pltpu.VMEM((CH,), jnp.float32),
            pltpu.VMEM((CH,), jnp.int32),
            pltpu.VMEM((NBINS_PAD,), jnp.int32),
        ],
        compiler_params=pltpu.CompilerParams(needs_layout_passes=False),
    )
    return kern(x2, tbl)


def _tc_body(codes_ref, gamma_ref, beta_ref, out_ref):
    c = pl.program_id(0)
    code = codes_ref[0]
    v = code - 1
    i3 = lax.shift_right_logical(v, 12)
    i4 = lax.shift_right_logical(v, 6) & (L - 1)
    i5 = v & (L - 1)
    m2 = (i3 * i3 + i4 * i4 + i5 * i5).astype(jnp.float32)
    mag = jnp.where(code > 0, jnp.sqrt(m2), 0.0)
    n = jnp.float32(B * N * N)
    s1 = jnp.sum(mag)
    s2 = jnp.sum(mag * mag)
    mean = s1 / n
    var = s2 / n - mean * mean
    inv = lax.rsqrt(var + 1e-5)
    g = gamma_ref[c]
    b = beta_ref[c]
    out_ref[0] = (mag - mean) * (inv * g) + b


def _tc_batchnorm(codes3, gamma, beta):
    return pl.pallas_call(
        _tc_body,
        out_shape=jax.ShapeDtypeStruct((C, B, NBINS_PAD), jnp.float32),
        grid=(C,),
        in_specs=[
            pl.BlockSpec((1, B, NBINS_PAD), lambda c: (c, 0, 0)),
            pl.BlockSpec(memory_space=pltpu.SMEM),
            pl.BlockSpec(memory_space=pltpu.SMEM),
        ],
        out_specs=pl.BlockSpec((1, B, NBINS_PAD), lambda c: (c, 0, 0)),
    )(codes3, gamma, beta)


def kernel(x, gamma, beta):
    x2 = x.reshape(B * C, V)
    tbl = jnp.asarray(_BIN_TABLE)
    codes = _sc_scatter(x2, tbl)
    codes3 = codes.reshape(C, B, NBINS_PAD)
    y = _tc_batchnorm(codes3, gamma, beta)
    return y.transpose(1, 0, 2)[:, :, : N * N].reshape(B, C, N, N)


# speedup vs baseline: 806.8274x; 1.1977x over previous
import functools

import jax
import jax.numpy as jnp
import numpy as np
from jax import lax
from jax.experimental import pallas as pl
from jax.experimental.pallas import tpu as pltpu
from jax.experimental.pallas import tpu_sc as plsc

B = 8
C = 4
L = 64
N = 90
BIN = 360 // N
V = L * L * L
NW = 32
NBINS_PAD = 8128
CH = 16384
NCHUNK = V // CH
LANES = 16


def _build_bin_table() -> np.ndarray:
    i3, i4, i5 = np.meshgrid(
        np.arange(L), np.arange(L), np.arange(L), indexing="ij"
    )
    i3 = i3.ravel().astype(np.float32)
    i4 = i4.ravel().astype(np.float32)
    i5 = i5.ravel().astype(np.float32)
    x_comp = np.sqrt(i4 * i4 + i5 * i5, dtype=np.float32)
    z_comp = np.sqrt(i4 * i4 + i3 * i3, dtype=np.float32)
    deg = np.float32(180.0)
    pi = np.float32(np.pi)
    binf = np.float32(BIN)
    ax = (np.arctan2(x_comp, i3) * deg / pi / binf).astype(np.int32)
    az = (np.arctan2(i5, z_comp) * deg / pi / binf).astype(np.int32)
    return (ax * np.int32(N) + az).astype(np.int32)


_BIN_TABLE = _build_bin_table()


UNROLL = 8


def _sc_body(x_hbm, tbl_hbm, out_hbm, xb0, xb1, tb0, tb1, acc, sem0, sem1):
    wid = lax.axis_index("s") * 2 + lax.axis_index("c")
    xrow = (wid & 7) * C + lax.shift_right_logical(wid, 3)

    xb = (xb0, xb1)
    tb = (tb0, tb1)
    sems = (sem0, sem1)

    def _issue(chunk):
        par = chunk & 1
        hx = pltpu.async_copy(
            x_hbm.at[xrow, pl.ds(chunk * CH, CH)], xb[par], sems[par]
        )
        ht = pltpu.async_copy(tbl_hbm.at[pl.ds(chunk * CH, CH)], tb[par], sems[par])
        return hx, ht

    pending = {0: _issue(0)}

    zero16 = jnp.zeros((LANES,), jnp.int32)

    def _zero(i, _):
        acc[pl.ds(i * LANES, LANES)] = zero16
        return _

    lax.fori_loop(0, NBINS_PAD // LANES, _zero, None)

    iota1 = lax.iota(jnp.int32, LANES) + 1

    for chunk in range(NCHUNK):
        hx, ht = pending.pop(chunk)
        hx.wait()
        ht.wait()
        if chunk + 1 < NCHUNK:
            pending[chunk + 1] = _issue(chunk + 1)
        par = chunk & 1
        xbuf, tbuf = xb[par], tb[par]
        base0 = chunk * CH

        def _step(i, _, xbuf=xbuf, tbuf=tbuf, base0=base0):
            ibase = i * (LANES * UNROLL)
            for j in range(UNROLL):
                off = ibase + j * LANES
                xv = xbuf[pl.ds(off, LANES)]
                iv = tbuf[pl.ds(off, LANES)]
                code = iota1 + (base0 + off)
                plsc.store_scatter(acc, [iv], code, mask=xv != 0.0)
            return _

        lax.fori_loop(0, CH // (LANES * UNROLL), _step, None)

    pltpu.sync_copy(acc, out_hbm.at[wid])


@functools.partial(jax.jit, static_argnames=())
def _sc_scatter(x2, tbl):
    mesh = plsc.VectorSubcoreMesh(core_axis_name="c", subcore_axis_name="s")
    kern = pl.kernel(
        _sc_body,
        out_type=jax.ShapeDtypeStruct((NW, NBINS_PAD), jnp.int32),
        mesh=mesh,
        scratch_types=[
            pltpu.VMEM((CH,), jnp.float32),
            pltpu.VMEM((CH,), jnp.float32),
            pltpu.VMEM((CH,), jnp.int32),
            pltpu.VMEM((CH,), jnp.int32),
            pltpu.VMEM((NBINS_PAD,), jnp.int32),
            pltpu.SemaphoreType.DMA,
            pltpu.SemaphoreType.DMA,
        ],
        compiler_params=pltpu.CompilerParams(needs_layout_passes=False),
    )
    return kern(x2, tbl)


def _tc_body(codes_ref, gamma_ref, beta_ref, out_ref):
    c = pl.program_id(0)
    code = codes_ref[0]
    v = code - 1
    i3 = lax.shift_right_logical(v, 12)
    i4 = lax.shift_right_logical(v, 6) & (L - 1)
    i5 = v & (L - 1)
    m2 = (i3 * i3 + i4 * i4 + i5 * i5).astype(jnp.float32)
    mag = jnp.where(code > 0, jnp.sqrt(m2), 0.0)
    n = jnp.float32(B * N * N)
    s1 = jnp.sum(mag)
    s2 = jnp.sum(mag * mag)
    mean = s1 / n
    var = s2 / n - mean * mean
    inv = lax.rsqrt(var + 1e-5)
    g = gamma_ref[c]
    b = beta_ref[c]
    out_ref[0] = (mag - mean) * (inv * g) + b


def _tc_batchnorm(codes3, gamma, beta):
    return pl.pallas_call(
        _tc_body,
        out_shape=jax.ShapeDtypeStruct((C, B, NBINS_PAD), jnp.float32),
        grid=(C,),
        in_specs=[
            pl.BlockSpec((1, B, NBINS_PAD), lambda c: (c, 0, 0)),
            pl.BlockSpec(memory_space=pltpu.SMEM),
            pl.BlockSpec(memory_space=pltpu.SMEM),
        ],
        out_specs=pl.BlockSpec((1, B, NBINS_PAD), lambda c: (c, 0, 0)),
    )(codes3, gamma, beta)


def kernel(x, gamma, beta):
    x2 = x.reshape(B * C, V)
    tbl = jnp.asarray(_BIN_TABLE)
    codes = _sc_scatter(x2, tbl)
    codes3 = codes.reshape(C, B, NBINS_PAD)
    y = _tc_batchnorm(codes3, gamma, beta)
    return y.transpose(1, 0, 2)[:, :, : N * N].reshape(B, C, N, N)


# speedup vs baseline: 1318.3306x; 1.6340x over previous
import functools

import jax
import jax.numpy as jnp
import numpy as np
from jax import lax
from jax.experimental import pallas as pl
from jax.experimental.pallas import tpu as pltpu
from jax.experimental.pallas import tpu_sc as plsc

B = 8
C = 4
L = 64
N = 90
BIN = 360 // N
V = L * L * L
NW = 32
NBINS_PAD = 8128
CH = 16384
NCHUNK = V // CH
LANES = 16


def _build_bin_table() -> np.ndarray:
    i3, i4, i5 = np.meshgrid(
        np.arange(L), np.arange(L), np.arange(L), indexing="ij"
    )
    i3 = i3.ravel().astype(np.float32)
    i4 = i4.ravel().astype(np.float32)
    i5 = i5.ravel().astype(np.float32)
    x_comp = np.sqrt(i4 * i4 + i5 * i5, dtype=np.float32)
    z_comp = np.sqrt(i4 * i4 + i3 * i3, dtype=np.float32)
    deg = np.float32(180.0)
    pi = np.float32(np.pi)
    binf = np.float32(BIN)
    ax = (np.arctan2(x_comp, i3) * deg / pi / binf).astype(np.int32)
    az = (np.arctan2(i5, z_comp) * deg / pi / binf).astype(np.int32)
    return (ax * np.int32(N) + az).astype(np.int32)


_BIN_TABLE = _build_bin_table()


UNROLL = 8


def _sc_body(x_hbm, tbl_hbm, out_hbm, xb0, xb1, tb0, tb1, acc, sem0, sem1):
    wid = lax.axis_index("s") * 2 + lax.axis_index("c")
    xrow = (wid & 7) * C + lax.shift_right_logical(wid, 3)

    xb = (xb0, xb1)
    tb = (tb0, tb1)
    sems = (sem0, sem1)

    def _issue(chunk):
        par = chunk & 1
        hx = pltpu.async_copy(
            x_hbm.at[xrow, pl.ds(chunk * CH, CH)],
            xb[par].at[pl.ds(0, CH)],
            sems[par],
        )
        ht = pltpu.async_copy(
            tbl_hbm.at[pl.ds(chunk * CH, CH)],
            tb[par].at[pl.ds(0, CH)],
            sems[par],
        )
        return hx, ht

    pending = {0: _issue(0)}

    zero16 = jnp.zeros((LANES,), jnp.int32)

    def _zero(i, _):
        acc[pl.ds(i * LANES, LANES)] = zero16
        return _

    lax.fori_loop(0, NBINS_PAD // LANES, _zero, None)

    iota1 = lax.iota(jnp.int32, LANES) + 1

    for chunk in range(NCHUNK):
        hx, ht = pending.pop(chunk)
        hx.wait()
        ht.wait()
        if chunk + 1 < NCHUNK:
            pending[chunk + 1] = _issue(chunk + 1)
        par = chunk & 1
        xbuf, tbuf = xb[par], tb[par]
        base0 = chunk * CH
        blk = LANES * UNROLL

        def _loads(off, xbuf=xbuf, tbuf=tbuf):
            vs = []
            for j in range(UNROLL):
                o = off + j * LANES
                vs.append(xbuf[pl.ds(o, LANES)])
                vs.append(tbuf[pl.ds(o, LANES)])
            return tuple(vs)

        def _step(i, carry, base0=base0, loads=_loads):
            vs = loads((i + 1) * blk)
            for j in range(UNROLL):
                xv = carry[2 * j]
                iv = carry[2 * j + 1]
                code = iota1 + (base0 + i * blk + j * LANES)
                plsc.store_scatter(acc, [iv], code, mask=xv > 0.5)
            return vs

        lax.fori_loop(0, CH // blk, _step, _loads(0))

    pltpu.sync_copy(acc, out_hbm.at[wid])


@functools.partial(jax.jit, static_argnames=())
def _sc_scatter(x2, tbl):
    mesh = plsc.VectorSubcoreMesh(core_axis_name="c", subcore_axis_name="s")
    kern = pl.kernel(
        _sc_body,
        out_type=jax.ShapeDtypeStruct((NW, NBINS_PAD), jnp.int32),
        mesh=mesh,
        scratch_types=[
            pltpu.VMEM((CH + LANES * UNROLL,), jnp.float32),
            pltpu.VMEM((CH + LANES * UNROLL,), jnp.float32),
            pltpu.VMEM((CH + LANES * UNROLL,), jnp.int32),
            pltpu.VMEM((CH + LANES * UNROLL,), jnp.int32),
            pltpu.VMEM((NBINS_PAD,), jnp.int32),
            pltpu.SemaphoreType.DMA,
            pltpu.SemaphoreType.DMA,
        ],
        compiler_params=pltpu.CompilerParams(needs_layout_passes=False),
    )
    return kern(x2, tbl)


def _tc_body(codes_ref, gamma_ref, beta_ref, out_ref):
    c = pl.program_id(0)
    code = codes_ref[0]
    v = code - 1
    i3 = lax.shift_right_logical(v, 12)
    i4 = lax.shift_right_logical(v, 6) & (L - 1)
    i5 = v & (L - 1)
    m2 = (i3 * i3 + i4 * i4 + i5 * i5).astype(jnp.float32)
    mag = jnp.where(code > 0, jnp.sqrt(m2), 0.0)
    n = jnp.float32(B * N * N)
    s1 = jnp.sum(mag)
    s2 = jnp.sum(mag * mag)
    mean = s1 / n
    var = s2 / n - mean * mean
    inv = lax.rsqrt(var + 1e-5)
    g = gamma_ref[c]
    b = beta_ref[c]
    out_ref[0] = (mag - mean) * (inv * g) + b


def _tc_batchnorm(codes3, gamma, beta):
    return pl.pallas_call(
        _tc_body,
        out_shape=jax.ShapeDtypeStruct((C, B, NBINS_PAD), jnp.float32),
        grid=(C,),
        in_specs=[
            pl.BlockSpec((1, B, NBINS_PAD), lambda c: (c, 0, 0)),
            pl.BlockSpec(memory_space=pltpu.SMEM),
            pl.BlockSpec(memory_space=pltpu.SMEM),
        ],
        out_specs=pl.BlockSpec((1, B, NBINS_PAD), lambda c: (c, 0, 0)),
    )(codes3, gamma, beta)


def kernel(x, gamma, beta):
    x2 = x.reshape(B * C, V)
    tbl = jnp.asarray(_BIN_TABLE)
    codes = _sc_scatter(x2, tbl)
    codes3 = codes.reshape(C, B, NBINS_PAD)
    y = _tc_batchnorm(codes3, gamma, beta)
    return y.transpose(1, 0, 2)[:, :, : N * N].reshape(B, C, N, N)
